# Initial kernel scaffold; baseline (speedup 1.0000x reference)
import jax
import jax.numpy as jnp
from jax.experimental import pallas as pl


def kernel(x, pos, edge_index, cell_offsets, Wemb, bemb, Wm1, bm1, Wm2, bm2, Wu1, bu1, Wu2, bu2, gamma, beta, Whead, bhead):
    raise NotImplementedError("write your pallas kernel here")


# speedup vs baseline: 1.9404x; 1.9404x over previous
import functools

import jax
import jax.numpy as jnp
from jax import lax
from jax.experimental import pallas as pl
from jax.experimental.pallas import tpu as pltpu
from jax.experimental.pallas import tpu_sc as plsc

N = 10000
E = 320000
H = 128
NS = 16
NC = 2
NW = NC * NS
EPT = E // NW
CH = 80
NCH = EPT // CH

F32 = jnp.float32


def _mesh():
    return plsc.VectorSubcoreMesh(core_axis_name="c", subcore_axis_name="s",
                                  num_cores=NC, num_subcores=NS)


_SC_PARAMS = pltpu.CompilerParams(needs_layout_passes=False)


def _sc_gather(table, src, dst):
    @functools.partial(
        pl.kernel,
        out_type=[jax.ShapeDtypeStruct((E, H), F32),
                  jax.ShapeDtypeStruct((E, H), F32)],
        mesh=_mesh(),
        scratch_types=[
            pltpu.VMEM((CH,), jnp.int32), pltpu.VMEM((CH,), jnp.int32),
            pltpu.VMEM((CH, H), F32), pltpu.VMEM((CH, H), F32),
            pltpu.SemaphoreType.DMA, pltpu.SemaphoreType.DMA,
        ],
        compiler_params=_SC_PARAMS,
    )
    def k(t_h, src_h, dst_h, hi_h, hj_h, didx, sidx, ib, jb, si, sj):
        cid = lax.axis_index("c")
        sid = lax.axis_index("s")
        tb = (cid * NS + sid) * EPT

        def chunk(j, _):
            base = tb + j * CH
            pltpu.sync_copy(dst_h.at[pl.ds(base, CH)], didx)
            pltpu.sync_copy(src_h.at[pl.ds(base, CH)], sidx)
            ci = pltpu.async_copy(t_h.at[didx], ib, si)
            cj = pltpu.async_copy(t_h.at[sidx], jb, sj)
            ci.wait()
            cj.wait()
            pltpu.sync_copy(ib, hi_h.at[pl.ds(base, CH)])
            pltpu.sync_copy(jb, hj_h.at[pl.ds(base, CH)])
            return 0
        lax.fori_loop(0, NCH, chunk, 0)

    return k(table, src, dst)


def _swish(v):
    return v * jax.nn.sigmoid(v)


def _tc_embed(x, Wemb, bemb2):
    def body(x_r, w_r, b_r, h_r):
        pre = jnp.dot(x_r[...], w_r[...], preferred_element_type=F32) + b_r[...]
        h_r[...] = _swish(pre)

    return pl.pallas_call(
        body, out_shape=jax.ShapeDtypeStruct((N, H), F32),
    )(x, Wemb, bemb2)


def _tc_edge(h_i, h_j, geom5, Wm1, bm1row, Wm2, bm2row):
    blk = 2560
    grid = E // blk

    def body(hi_r, hj_r, g_r, w1_r, b1_r, w2_r, b2_r, o_r):
        m = jnp.concatenate([hi_r[...], hj_r[...], g_r[...]], axis=-1)
        m1 = _swish(jnp.dot(m, w1_r[...], preferred_element_type=F32)
                    + b1_r[...])
        o_r[...] = _swish(jnp.dot(m1, w2_r[...], preferred_element_type=F32)
                          + b2_r[...])

    return pl.pallas_call(
        body,
        grid=(grid,),
        in_specs=[
            pl.BlockSpec((blk, H), lambda i: (i, 0)),
            pl.BlockSpec((blk, H), lambda i: (i, 0)),
            pl.BlockSpec((blk, 5), lambda i: (i, 0)),
            pl.BlockSpec((2 * H + 5, H), lambda i: (0, 0)),
            pl.BlockSpec((1, H), lambda i: (0, 0)),
            pl.BlockSpec((H, H), lambda i: (0, 0)),
            pl.BlockSpec((1, H), lambda i: (0, 0)),
        ],
        out_specs=pl.BlockSpec((blk, H), lambda i: (i, 0)),
        out_shape=jax.ShapeDtypeStruct((E, H), F32),
    )(h_i, h_j, geom5, Wm1, bm1row, Wm2, bm2row)


def _tc_update(h, agg, Wu1, bu1r, Wu2, bu2r):
    def body(h_r, a_r, w1_r, b1_r, w2_r, b2_r, v_r):
        u = jnp.concatenate([h_r[...], a_r[...]], axis=-1)
        u = _swish(jnp.dot(u, w1_r[...], preferred_element_type=F32)
                   + b1_r[...])
        v_r[...] = jnp.dot(u, w2_r[...], preferred_element_type=F32) + b2_r[...]

    return pl.pallas_call(
        body, out_shape=jax.ShapeDtypeStruct((N, H), F32),
    )(h, agg, Wu1, bu1r, Wu2, bu2r)


def _tc_bn(v, mean, var, gam, bet):
    def body(v_r, m_r, s_r, g_r, b_r, o_r):
        o_r[...] = ((v_r[...] - m_r[...]) / jnp.sqrt(s_r[...] + 1e-5)
                    * g_r[...] + b_r[...])

    return pl.pallas_call(
        body, out_shape=jax.ShapeDtypeStruct((N, H), F32),
    )(v, mean, var, gam, bet)


def _tc_head(h, Whead, bhead11):
    def body(h_r, w_r, b_r, o_r):
        o_r[...] = (jnp.dot(h_r[...], w_r[...], preferred_element_type=F32)
                    + b_r[...])

    return pl.pallas_call(
        body, out_shape=jax.ShapeDtypeStruct((N, 1), F32),
    )(h, Whead, bhead11)


def kernel(x, pos, edge_index, cell_offsets, Wemb, bemb, Wm1, bm1, Wm2, bm2,
           Wu1, bu1, Wu2, bu2, gamma, beta, Whead, bhead):
    src = edge_index[0].astype(jnp.int32)
    dst = edge_index[1].astype(jnp.int32)
    nl = Wm1.shape[0]

    rel = (jnp.take(pos, dst, axis=0) - jnp.take(pos, src, axis=0)
           + cell_offsets)
    dist = jnp.linalg.norm(rel, axis=-1, keepdims=True)
    sh = jnp.concatenate([jnp.ones(rel.shape[:-1] + (1,), rel.dtype),
                          jnp.sqrt(3.0) * rel], axis=-1)
    geom5 = jnp.concatenate([dist, sh], axis=-1)
    cnt = jax.ops.segment_sum(jnp.ones((E, 1), F32), dst, num_segments=N)
    cdiv = jnp.maximum(cnt, 1.0)

    h = _tc_embed(x, Wemb, bemb.reshape(1, H))
    for l in range(nl):
        h_i, h_j = _sc_gather(h, src, dst)
        m2 = _tc_edge(h_i, h_j, geom5, Wm1[l], bm1[l].reshape(1, H),
                      Wm2[l], bm2[l].reshape(1, H))
        agg = jax.ops.segment_sum(m2, dst, num_segments=N) / cdiv
        v = _tc_update(h, agg, Wu1[l], bu1[l].reshape(1, H),
                       Wu2[l], bu2[l].reshape(1, H))
        mean = jnp.mean(v, axis=0, keepdims=True)
        var = jnp.var(v, axis=0, keepdims=True)
        h = _tc_bn(v, mean, var, gamma[l].reshape(1, H), beta[l].reshape(1, H))

    ne = _tc_head(h, Whead, bhead.reshape(1, 1))
    return jnp.sum(ne, axis=0)


# speedup vs baseline: 2.4136x; 1.2439x over previous
import functools

import jax
import jax.numpy as jnp
from jax import lax
from jax.experimental import pallas as pl
from jax.experimental.pallas import tpu as pltpu
from jax.experimental.pallas import tpu_sc as plsc

N = 10000
E = 320000
H = 128
NS = 16
NC = 2
NW = NC * NS
EPT = E // NW
CH = 80
NCH = EPT // CH

F32 = jnp.float32


def _mesh():
    return plsc.VectorSubcoreMesh(core_axis_name="c", subcore_axis_name="s",
                                  num_cores=NC, num_subcores=NS)


_SC_PARAMS = pltpu.CompilerParams(needs_layout_passes=False)


def _sc_geom(posx, posy, posz, src, dst, cox, coy, coz):
    @functools.partial(
        pl.kernel,
        out_type=jax.ShapeDtypeStruct((E * 8,), F32),
        mesh=_mesh(),
        scratch_types=[
            pltpu.VMEM((N,), F32), pltpu.VMEM((N,), F32), pltpu.VMEM((N,), F32),
            pltpu.VMEM((CH,), jnp.int32), pltpu.VMEM((CH,), jnp.int32),
            pltpu.VMEM((CH,), F32), pltpu.VMEM((CH,), F32), pltpu.VMEM((CH,), F32),
            pltpu.VMEM((CH * 8,), F32),
        ],
        compiler_params=_SC_PARAMS,
    )
    def k(posx_h, posy_h, posz_h, src_h, dst_h, cox_h, coy_h, coz_h, e8_h,
          px, py, pz, sidx, didx, cxv, cyv, czv, e8b):
        cid = lax.axis_index("c")
        sid = lax.axis_index("s")
        tb = (cid * NS + sid) * EPT
        pltpu.sync_copy(posx_h, px)
        pltpu.sync_copy(posy_h, py)
        pltpu.sync_copy(posz_h, pz)
        zeros16 = jnp.zeros((16,), F32)

        def initz(i, _):
            e8b[pl.ds(i * 16, 16)] = zeros16
            return 0
        lax.fori_loop(0, CH * 8 // 16, initz, 0)

        iot = lax.iota(jnp.int32, 16)

        def chunk(j, _):
            base = tb + j * CH
            pltpu.sync_copy(dst_h.at[pl.ds(base, CH)], didx)
            pltpu.sync_copy(src_h.at[pl.ds(base, CH)], sidx)
            pltpu.sync_copy(cox_h.at[pl.ds(base, CH)], cxv)
            pltpu.sync_copy(coy_h.at[pl.ds(base, CH)], cyv)
            pltpu.sync_copy(coz_h.at[pl.ds(base, CH)], czv)
            for i in range(CH // 16):
                d16 = didx[pl.ds(i * 16, 16)]
                s16 = sidx[pl.ds(i * 16, 16)]
                rx = (plsc.load_gather(px, [d16]) - plsc.load_gather(px, [s16])
                      + cxv[pl.ds(i * 16, 16)])
                ry = (plsc.load_gather(py, [d16]) - plsc.load_gather(py, [s16])
                      + cyv[pl.ds(i * 16, 16)])
                rz = (plsc.load_gather(pz, [d16]) - plsc.load_gather(pz, [s16])
                      + czv[pl.ds(i * 16, 16)])
                fb = (iot + i * 16) * 8
                plsc.store_scatter(e8b, [fb], rx)
                plsc.store_scatter(e8b, [fb + 1], ry)
                plsc.store_scatter(e8b, [fb + 2], rz)
            pltpu.sync_copy(e8b, e8_h.at[pl.ds(base * 8, CH * 8)])
            return 0
        lax.fori_loop(0, NCH, chunk, 0)

    return k(posx, posy, posz, src, dst, cox, coy, coz)


def _sc_gather(table, src, dst):
    @functools.partial(
        pl.kernel,
        out_type=[jax.ShapeDtypeStruct((E, H), F32),
                  jax.ShapeDtypeStruct((E, H), F32)],
        mesh=_mesh(),
        scratch_types=[
            pltpu.VMEM((2, CH), jnp.int32), pltpu.VMEM((2, CH), jnp.int32),
            pltpu.VMEM((2, CH, H), F32), pltpu.VMEM((2, CH, H), F32),
            pltpu.SemaphoreType.DMA, pltpu.SemaphoreType.DMA,
            pltpu.SemaphoreType.DMA, pltpu.SemaphoreType.DMA,
        ],
        compiler_params=_SC_PARAMS,
    )
    def k(t_h, src_h, dst_h, hi_h, hj_h, didx, sidx, ib, jb, si0, sj0, si1, sj1):
        cid = lax.axis_index("c")
        sid = lax.axis_index("s")
        tb = (cid * NS + sid) * EPT
        sems = ((si0, sj0), (si1, sj1))

        def fire(j, b):
            base = tb + j * CH
            pltpu.sync_copy(dst_h.at[pl.ds(base, CH)], didx.at[b])
            pltpu.sync_copy(src_h.at[pl.ds(base, CH)], sidx.at[b])
            ci = pltpu.async_copy(t_h.at[didx.at[b]], ib.at[b], sems[b][0])
            cj = pltpu.async_copy(t_h.at[sidx.at[b]], jb.at[b], sems[b][1])
            return ci, cj

        def drain(j, b, ci, cj):
            base = tb + j * CH
            ci.wait()
            cj.wait()
            pltpu.sync_copy(ib.at[b], hi_h.at[pl.ds(base, CH)])
            pltpu.sync_copy(jb.at[b], hj_h.at[pl.ds(base, CH)])

        def step(k2, _):
            j0 = k2 * 2
            c0 = fire(j0, 0)
            c1 = fire(j0 + 1, 1)
            drain(j0, 0, *c0)
            drain(j0 + 1, 1, *c1)
            return 0
        lax.fori_loop(0, NCH // 2, step, 0)
        cz = fire(NCH - 1, 0)
        drain(NCH - 1, 0, *cz)

    return k(table, src, dst)


def _swish(v):
    return v * jax.nn.sigmoid(v)


def _tc_embed(x, Wemb, bemb2):
    def body(x_r, w_r, b_r, h_r):
        pre = jnp.dot(x_r[...], w_r[...], preferred_element_type=F32) + b_r[...]
        h_r[...] = _swish(pre)

    return pl.pallas_call(
        body, out_shape=jax.ShapeDtypeStruct((N, H), F32),
    )(x, Wemb, bemb2)


def _tc_edge(h_i, h_j, geom5, Wm1, bm1row, Wm2, bm2row):
    blk = 2560
    grid = E // blk

    def body(hi_r, hj_r, g_r, w1_r, b1_r, w2_r, b2_r, o_r):
        m = jnp.concatenate([hi_r[...], hj_r[...], g_r[...]], axis=-1)
        m1 = _swish(jnp.dot(m, w1_r[...], preferred_element_type=F32)
                    + b1_r[...])
        o_r[...] = _swish(jnp.dot(m1, w2_r[...], preferred_element_type=F32)
                          + b2_r[...])

    return pl.pallas_call(
        body,
        grid=(grid,),
        in_specs=[
            pl.BlockSpec((blk, H), lambda i: (i, 0)),
            pl.BlockSpec((blk, H), lambda i: (i, 0)),
            pl.BlockSpec((blk, 5), lambda i: (i, 0)),
            pl.BlockSpec((2 * H + 5, H), lambda i: (0, 0)),
            pl.BlockSpec((1, H), lambda i: (0, 0)),
            pl.BlockSpec((H, H), lambda i: (0, 0)),
            pl.BlockSpec((1, H), lambda i: (0, 0)),
        ],
        out_specs=pl.BlockSpec((blk, H), lambda i: (i, 0)),
        out_shape=jax.ShapeDtypeStruct((E, H), F32),
    )(h_i, h_j, geom5, Wm1, bm1row, Wm2, bm2row)


def _tc_update(h, segsum, cnt, Wu1, bu1r, Wu2, bu2r):
    def body(h_r, s_r, c_r, w1_r, b1_r, w2_r, b2_r, v_r):
        agg = s_r[...] / jnp.maximum(c_r[...], 1.0)
        u = jnp.concatenate([h_r[...], agg], axis=-1)
        u = _swish(jnp.dot(u, w1_r[...], preferred_element_type=F32)
                   + b1_r[...])
        v_r[...] = jnp.dot(u, w2_r[...], preferred_element_type=F32) + b2_r[...]

    return pl.pallas_call(
        body, out_shape=jax.ShapeDtypeStruct((N, H), F32),
    )(h, segsum, cnt, Wu1, bu1r, Wu2, bu2r)


def _tc_bn(v, mean, var, gam, bet):
    def body(v_r, m_r, s_r, g_r, b_r, o_r):
        o_r[...] = ((v_r[...] - m_r[...]) / jnp.sqrt(s_r[...] + 1e-5)
                    * g_r[...] + b_r[...])

    return pl.pallas_call(
        body, out_shape=jax.ShapeDtypeStruct((N, H), F32),
    )(v, mean, var, gam, bet)


def _tc_head(h, Whead, bhead11):
    def body(h_r, w_r, b_r, o_r):
        o_r[...] = (jnp.dot(h_r[...], w_r[...], preferred_element_type=F32)
                    + b_r[...])

    return pl.pallas_call(
        body, out_shape=jax.ShapeDtypeStruct((N, 1), F32),
    )(h, Whead, bhead11)


def kernel(x, pos, edge_index, cell_offsets, Wemb, bemb, Wm1, bm1, Wm2, bm2,
           Wu1, bu1, Wu2, bu2, gamma, beta, Whead, bhead):
    src = edge_index[0].astype(jnp.int32)
    dst = edge_index[1].astype(jnp.int32)
    nl = Wm1.shape[0]

    e8 = _sc_geom(pos[:, 0], pos[:, 1], pos[:, 2], src, dst,
                  cell_offsets[:, 0], cell_offsets[:, 1],
                  cell_offsets[:, 2]).reshape(E, 8)
    rel = e8[:, :3]
    dist = jnp.linalg.norm(rel, axis=-1, keepdims=True)
    sh = jnp.concatenate([jnp.ones(rel.shape[:-1] + (1,), rel.dtype),
                          jnp.sqrt(3.0) * rel], axis=-1)
    geom5 = jnp.concatenate([dist, sh], axis=-1)
    cnt = jax.ops.segment_sum(jnp.ones((E, 1), F32), dst, num_segments=N)

    h = _tc_embed(x, Wemb, bemb.reshape(1, H))
    for l in range(nl):
        h_i, h_j = _sc_gather(h, src, dst)
        m2 = _tc_edge(h_i, h_j, geom5, Wm1[l], bm1[l].reshape(1, H),
                      Wm2[l], bm2[l].reshape(1, H))
        segsum = jax.ops.segment_sum(m2, dst, num_segments=N)
        v = _tc_update(h, segsum, cnt, Wu1[l], bu1[l].reshape(1, H),
                       Wu2[l], bu2[l].reshape(1, H))
        mean = jnp.mean(v, axis=0, keepdims=True)
        var = jnp.var(v, axis=0, keepdims=True)
        h = _tc_bn(v, mean, var, gamma[l].reshape(1, H), beta[l].reshape(1, H))

    ne = _tc_head(h, Whead, bhead.reshape(1, 1))
    return jnp.sum(ne, axis=0)


# speedup vs baseline: 2.4585x; 1.0186x over previous
import functools

import jax
import jax.numpy as jnp
from jax import lax
from jax.experimental import pallas as pl
from jax.experimental.pallas import tpu as pltpu
from jax.experimental.pallas import tpu_sc as plsc

N = 10000
E = 320000
H = 128
NS = 16
NC = 2
NW = NC * NS
EPT = E // NW
CH = 80
NCH = EPT // CH

F32 = jnp.float32


def _mesh():
    return plsc.VectorSubcoreMesh(core_axis_name="c", subcore_axis_name="s",
                                  num_cores=NC, num_subcores=NS)


_SC_PARAMS = pltpu.CompilerParams(needs_layout_passes=False)


def _sc_geom(posx, posy, posz, src, dst, cox, coy, coz):
    CHG = 400
    NCHG = EPT // CHG

    @functools.partial(
        pl.kernel,
        out_type=jax.ShapeDtypeStruct((E * 8,), F32),
        mesh=_mesh(),
        scratch_types=[
            pltpu.VMEM((N,), F32), pltpu.VMEM((N,), F32), pltpu.VMEM((N,), F32),
            pltpu.VMEM((CHG,), jnp.int32), pltpu.VMEM((CHG,), jnp.int32),
            pltpu.VMEM((CHG,), F32), pltpu.VMEM((CHG,), F32), pltpu.VMEM((CHG,), F32),
            pltpu.VMEM((CHG * 8,), F32),
        ],
        compiler_params=_SC_PARAMS,
    )
    def k(posx_h, posy_h, posz_h, src_h, dst_h, cox_h, coy_h, coz_h, e8_h,
          px, py, pz, sidx, didx, cxv, cyv, czv, e8b):
        cid = lax.axis_index("c")
        sid = lax.axis_index("s")
        tb = (cid * NS + sid) * EPT
        pltpu.sync_copy(posx_h, px)
        pltpu.sync_copy(posy_h, py)
        pltpu.sync_copy(posz_h, pz)
        zeros16 = jnp.zeros((16,), F32)

        def initz(i, _):
            e8b[pl.ds(i * 16, 16)] = zeros16
            return 0
        lax.fori_loop(0, CHG * 8 // 16, initz, 0)

        iot = lax.iota(jnp.int32, 16)

        def chunk(j, _):
            base = tb + j * CHG
            pltpu.sync_copy(dst_h.at[pl.ds(base, CHG)], didx)
            pltpu.sync_copy(src_h.at[pl.ds(base, CHG)], sidx)
            pltpu.sync_copy(cox_h.at[pl.ds(base, CHG)], cxv)
            pltpu.sync_copy(coy_h.at[pl.ds(base, CHG)], cyv)
            pltpu.sync_copy(coz_h.at[pl.ds(base, CHG)], czv)

            def sub16(i, _):
                o = i * 16
                d16 = didx[pl.ds(o, 16)]
                s16 = sidx[pl.ds(o, 16)]
                rx = (plsc.load_gather(px, [d16]) - plsc.load_gather(px, [s16])
                      + cxv[pl.ds(o, 16)])
                ry = (plsc.load_gather(py, [d16]) - plsc.load_gather(py, [s16])
                      + cyv[pl.ds(o, 16)])
                rz = (plsc.load_gather(pz, [d16]) - plsc.load_gather(pz, [s16])
                      + czv[pl.ds(o, 16)])
                fb = (iot + o) * 8
                plsc.store_scatter(e8b, [fb], rx)
                plsc.store_scatter(e8b, [fb + 1], ry)
                plsc.store_scatter(e8b, [fb + 2], rz)
                return 0
            lax.fori_loop(0, CHG // 16, sub16, 0)
            pltpu.sync_copy(e8b, e8_h.at[pl.ds(base * 8, CHG * 8)])
            return 0
        lax.fori_loop(0, NCHG, chunk, 0)

    return k(posx, posy, posz, src, dst, cox, coy, coz)


def _sc_gather(table, src, dst):
    @functools.partial(
        pl.kernel,
        out_type=[jax.ShapeDtypeStruct((E, H), F32),
                  jax.ShapeDtypeStruct((E, H), F32)],
        mesh=_mesh(),
        scratch_types=[
            pltpu.VMEM((EPT,), jnp.int32), pltpu.VMEM((EPT,), jnp.int32),
            pltpu.VMEM((2, CH, H), F32), pltpu.VMEM((2, CH, H), F32),
            pltpu.SemaphoreType.DMA, pltpu.SemaphoreType.DMA,
            pltpu.SemaphoreType.DMA, pltpu.SemaphoreType.DMA,
        ],
        compiler_params=_SC_PARAMS,
    )
    def k(t_h, src_h, dst_h, hi_h, hj_h, didx, sidx, ib, jb, si0, sj0, si1, sj1):
        cid = lax.axis_index("c")
        sid = lax.axis_index("s")
        tb = (cid * NS + sid) * EPT
        sems = ((si0, sj0), (si1, sj1))
        pltpu.sync_copy(dst_h.at[pl.ds(tb, EPT)], didx)
        pltpu.sync_copy(src_h.at[pl.ds(tb, EPT)], sidx)

        def fire(j, b):
            ci = pltpu.async_copy(t_h.at[didx.at[pl.ds(j * CH, CH)]],
                                  ib.at[b], sems[b][0])
            cj = pltpu.async_copy(t_h.at[sidx.at[pl.ds(j * CH, CH)]],
                                  jb.at[b], sems[b][1])
            return ci, cj

        def drain(j, b, ci, cj):
            base = tb + j * CH
            ci.wait()
            cj.wait()
            pltpu.sync_copy(ib.at[b], hi_h.at[pl.ds(base, CH)])
            pltpu.sync_copy(jb.at[b], hj_h.at[pl.ds(base, CH)])

        def step(k2, _):
            j0 = k2 * 2
            c0 = fire(j0, 0)
            c1 = fire(j0 + 1, 1)
            drain(j0, 0, *c0)
            drain(j0 + 1, 1, *c1)
            return 0
        lax.fori_loop(0, NCH // 2, step, 0)
        cz = fire(NCH - 1, 0)
        drain(NCH - 1, 0, *cz)

    return k(table, src, dst)


def _swish(v):
    return v * jax.nn.sigmoid(v)


def _tc_embed(x, Wemb, bemb2):
    def body(x_r, w_r, b_r, h_r):
        pre = jnp.dot(x_r[...], w_r[...], preferred_element_type=F32) + b_r[...]
        h_r[...] = _swish(pre)

    return pl.pallas_call(
        body, out_shape=jax.ShapeDtypeStruct((N, H), F32),
    )(x, Wemb, bemb2)


def _tc_edge(h_i, h_j, geom5, Wm1, bm1row, Wm2, bm2row):
    blk = 2560
    grid = E // blk

    def body(hi_r, hj_r, g_r, w1_r, b1_r, w2_r, b2_r, o_r):
        m = jnp.concatenate([hi_r[...], hj_r[...], g_r[...]], axis=-1)
        m1 = _swish(jnp.dot(m, w1_r[...], preferred_element_type=F32)
                    + b1_r[...])
        o_r[...] = _swish(jnp.dot(m1, w2_r[...], preferred_element_type=F32)
                          + b2_r[...])

    return pl.pallas_call(
        body,
        grid=(grid,),
        in_specs=[
            pl.BlockSpec((blk, H), lambda i: (i, 0)),
            pl.BlockSpec((blk, H), lambda i: (i, 0)),
            pl.BlockSpec((blk, 5), lambda i: (i, 0)),
            pl.BlockSpec((2 * H + 5, H), lambda i: (0, 0)),
            pl.BlockSpec((1, H), lambda i: (0, 0)),
            pl.BlockSpec((H, H), lambda i: (0, 0)),
            pl.BlockSpec((1, H), lambda i: (0, 0)),
        ],
        out_specs=pl.BlockSpec((blk, H), lambda i: (i, 0)),
        out_shape=jax.ShapeDtypeStruct((E, H), F32),
    )(h_i, h_j, geom5, Wm1, bm1row, Wm2, bm2row)


def _tc_update(h, segsum, cnt, Wu1, bu1r, Wu2, bu2r):
    def body(h_r, s_r, c_r, w1_r, b1_r, w2_r, b2_r, v_r):
        agg = s_r[...] / jnp.maximum(c_r[...], 1.0)
        u = jnp.concatenate([h_r[...], agg], axis=-1)
        u = _swish(jnp.dot(u, w1_r[...], preferred_element_type=F32)
                   + b1_r[...])
        v_r[...] = jnp.dot(u, w2_r[...], preferred_element_type=F32) + b2_r[...]

    return pl.pallas_call(
        body, out_shape=jax.ShapeDtypeStruct((N, H), F32),
    )(h, segsum, cnt, Wu1, bu1r, Wu2, bu2r)


def _tc_bn(v, mean, var, gam, bet):
    def body(v_r, m_r, s_r, g_r, b_r, o_r):
        o_r[...] = ((v_r[...] - m_r[...]) / jnp.sqrt(s_r[...] + 1e-5)
                    * g_r[...] + b_r[...])

    return pl.pallas_call(
        body, out_shape=jax.ShapeDtypeStruct((N, H), F32),
    )(v, mean, var, gam, bet)


def _tc_head(h, Whead, bhead11):
    def body(h_r, w_r, b_r, o_r):
        o_r[...] = (jnp.dot(h_r[...], w_r[...], preferred_element_type=F32)
                    + b_r[...])

    return pl.pallas_call(
        body, out_shape=jax.ShapeDtypeStruct((N, 1), F32),
    )(h, Whead, bhead11)


def kernel(x, pos, edge_index, cell_offsets, Wemb, bemb, Wm1, bm1, Wm2, bm2,
           Wu1, bu1, Wu2, bu2, gamma, beta, Whead, bhead):
    src = edge_index[0].astype(jnp.int32)
    dst = edge_index[1].astype(jnp.int32)
    nl = Wm1.shape[0]

    e8 = _sc_geom(pos[:, 0], pos[:, 1], pos[:, 2], src, dst,
                  cell_offsets[:, 0], cell_offsets[:, 1],
                  cell_offsets[:, 2]).reshape(E, 8)
    rel = e8[:, :3]
    dist = jnp.linalg.norm(rel, axis=-1, keepdims=True)
    sh = jnp.concatenate([jnp.ones(rel.shape[:-1] + (1,), rel.dtype),
                          jnp.sqrt(3.0) * rel], axis=-1)
    geom5 = jnp.concatenate([dist, sh], axis=-1)
    cnt = jax.ops.segment_sum(jnp.ones((E, 1), F32), dst, num_segments=N)

    h = _tc_embed(x, Wemb, bemb.reshape(1, H))
    for l in range(nl):
        h_i, h_j = _sc_gather(h, src, dst)
        m2 = _tc_edge(h_i, h_j, geom5, Wm1[l], bm1[l].reshape(1, H),
                      Wm2[l], bm2[l].reshape(1, H))
        segsum = jax.ops.segment_sum(m2, dst, num_segments=N)
        v = _tc_update(h, segsum, cnt, Wu1[l], bu1[l].reshape(1, H),
                       Wu2[l], bu2[l].reshape(1, H))
        mean = jnp.mean(v, axis=0, keepdims=True)
        var = jnp.var(v, axis=0, keepdims=True)
        h = _tc_bn(v, mean, var, gamma[l].reshape(1, H), beta[l].reshape(1, H))

    ne = _tc_head(h, Whead, bhead.reshape(1, 1))
    return jnp.sum(ne, axis=0)


# speedup vs baseline: 2.4611x; 1.0011x over previous
import functools

import jax
import jax.numpy as jnp
from jax import lax
from jax.experimental import pallas as pl
from jax.experimental.pallas import tpu as pltpu
from jax.experimental.pallas import tpu_sc as plsc

N = 10000
E = 320000
H = 128
NS = 16
NC = 2
NW = NC * NS
EPT = E // NW
CH = 80
NCH = EPT // CH

F32 = jnp.float32


def _mesh():
    return plsc.VectorSubcoreMesh(core_axis_name="c", subcore_axis_name="s",
                                  num_cores=NC, num_subcores=NS)


_SC_PARAMS = pltpu.CompilerParams(needs_layout_passes=False)


def _sc_geom(posx, posy, posz, src, dst, cox, coy, coz):
    CHG = 400
    NCHG = EPT // CHG

    @functools.partial(
        pl.kernel,
        out_type=jax.ShapeDtypeStruct((E * 8,), F32),
        mesh=_mesh(),
        scratch_types=[
            pltpu.VMEM((N,), F32), pltpu.VMEM((N,), F32), pltpu.VMEM((N,), F32),
            pltpu.VMEM((CHG,), jnp.int32), pltpu.VMEM((CHG,), jnp.int32),
            pltpu.VMEM((CHG,), F32), pltpu.VMEM((CHG,), F32), pltpu.VMEM((CHG,), F32),
            pltpu.VMEM((CHG * 8,), F32),
        ],
        compiler_params=_SC_PARAMS,
    )
    def k(posx_h, posy_h, posz_h, src_h, dst_h, cox_h, coy_h, coz_h, e8_h,
          px, py, pz, sidx, didx, cxv, cyv, czv, e8b):
        cid = lax.axis_index("c")
        sid = lax.axis_index("s")
        tb = (cid * NS + sid) * EPT
        pltpu.sync_copy(posx_h, px)
        pltpu.sync_copy(posy_h, py)
        pltpu.sync_copy(posz_h, pz)
        zeros16 = jnp.zeros((16,), F32)

        def initz(i, _):
            e8b[pl.ds(i * 16, 16)] = zeros16
            return 0
        lax.fori_loop(0, CHG * 8 // 16, initz, 0)

        iot = lax.iota(jnp.int32, 16)

        def chunk(j, _):
            base = tb + j * CHG
            pltpu.sync_copy(dst_h.at[pl.ds(base, CHG)], didx)
            pltpu.sync_copy(src_h.at[pl.ds(base, CHG)], sidx)
            pltpu.sync_copy(cox_h.at[pl.ds(base, CHG)], cxv)
            pltpu.sync_copy(coy_h.at[pl.ds(base, CHG)], cyv)
            pltpu.sync_copy(coz_h.at[pl.ds(base, CHG)], czv)

            def sub16(i, _):
                o = i * 16
                d16 = didx[pl.ds(o, 16)]
                s16 = sidx[pl.ds(o, 16)]
                rx = (plsc.load_gather(px, [d16]) - plsc.load_gather(px, [s16])
                      + cxv[pl.ds(o, 16)])
                ry = (plsc.load_gather(py, [d16]) - plsc.load_gather(py, [s16])
                      + cyv[pl.ds(o, 16)])
                rz = (plsc.load_gather(pz, [d16]) - plsc.load_gather(pz, [s16])
                      + czv[pl.ds(o, 16)])
                fb = (iot + o) * 8
                plsc.store_scatter(e8b, [fb], rx)
                plsc.store_scatter(e8b, [fb + 1], ry)
                plsc.store_scatter(e8b, [fb + 2], rz)
                return 0
            lax.fori_loop(0, CHG // 16, sub16, 0)
            pltpu.sync_copy(e8b, e8_h.at[pl.ds(base * 8, CHG * 8)])
            return 0
        lax.fori_loop(0, NCHG, chunk, 0)

    return k(posx, posy, posz, src, dst, cox, coy, coz)


def _sc_gather(table, src, dst):
    @functools.partial(
        pl.kernel,
        out_type=[jax.ShapeDtypeStruct((E, H), F32),
                  jax.ShapeDtypeStruct((E, H), F32)],
        mesh=_mesh(),
        scratch_types=[
            pltpu.VMEM((EPT,), jnp.int32), pltpu.VMEM((EPT,), jnp.int32),
            pltpu.VMEM((2, CH, H), F32), pltpu.VMEM((2, CH, H), F32),
            pltpu.SemaphoreType.DMA, pltpu.SemaphoreType.DMA,
            pltpu.SemaphoreType.DMA, pltpu.SemaphoreType.DMA,
        ],
        compiler_params=_SC_PARAMS,
    )
    def k(t_h, src_h, dst_h, hi_h, hj_h, didx, sidx, ib, jb, si0, sj0, si1, sj1):
        cid = lax.axis_index("c")
        sid = lax.axis_index("s")
        tb = (cid * NS + sid) * EPT
        sems = ((si0, sj0), (si1, sj1))
        pltpu.sync_copy(dst_h.at[pl.ds(tb, EPT)], didx)
        pltpu.sync_copy(src_h.at[pl.ds(tb, EPT)], sidx)

        def fire(j, b):
            ci = pltpu.async_copy(t_h.at[didx.at[pl.ds(j * CH, CH)]],
                                  ib.at[b], sems[b][0])
            cj = pltpu.async_copy(t_h.at[sidx.at[pl.ds(j * CH, CH)]],
                                  jb.at[b], sems[b][1])
            return ci, cj

        def drain(j, b, ci, cj):
            base = tb + j * CH
            ci.wait()
            cj.wait()
            pltpu.sync_copy(ib.at[b], hi_h.at[pl.ds(base, CH)])
            pltpu.sync_copy(jb.at[b], hj_h.at[pl.ds(base, CH)])

        def step(k2, _):
            j0 = k2 * 2
            c0 = fire(j0, 0)
            c1 = fire(j0 + 1, 1)
            drain(j0, 0, *c0)
            drain(j0 + 1, 1, *c1)
            return 0
        lax.fori_loop(0, NCH // 2, step, 0)
        cz = fire(NCH - 1, 0)
        drain(NCH - 1, 0, *cz)

    return k(table, src, dst)


def _swish(v):
    return v * jax.nn.sigmoid(v)


def _tc_embed(x, Wemb, bemb2):
    def body(x_r, w_r, b_r, h_r):
        pre = jnp.dot(x_r[...], w_r[...], preferred_element_type=F32) + b_r[...]
        h_r[...] = _swish(pre)

    return pl.pallas_call(
        body, out_shape=jax.ShapeDtypeStruct((N, H), F32),
    )(x, Wemb, bemb2)


def _tc_edge(h_i, h_j, geom5, Wm1, bm1row, Wm2, bm2row):
    blk = 2560
    grid = E // blk

    def body(hi_r, hj_r, g_r, w1_r, b1_r, w2_r, b2_r, o_r):
        m = jnp.concatenate([hi_r[...], hj_r[...], g_r[...]], axis=-1)
        m1 = _swish(jnp.dot(m, w1_r[...], preferred_element_type=F32)
                    + b1_r[...])
        o_r[...] = _swish(jnp.dot(m1, w2_r[...], preferred_element_type=F32)
                          + b2_r[...])

    return pl.pallas_call(
        body,
        grid=(grid,),
        in_specs=[
            pl.BlockSpec((blk, H), lambda i: (i, 0)),
            pl.BlockSpec((blk, H), lambda i: (i, 0)),
            pl.BlockSpec((blk, 5), lambda i: (i, 0)),
            pl.BlockSpec((2 * H + 5, H), lambda i: (0, 0)),
            pl.BlockSpec((1, H), lambda i: (0, 0)),
            pl.BlockSpec((H, H), lambda i: (0, 0)),
            pl.BlockSpec((1, H), lambda i: (0, 0)),
        ],
        out_specs=pl.BlockSpec((blk, H), lambda i: (i, 0)),
        out_shape=jax.ShapeDtypeStruct((E, H), F32),
    )(h_i, h_j, geom5, Wm1, bm1row, Wm2, bm2row)


def _tc_update(h, segsum, cnt, Wu1, bu1r, Wu2, bu2r):
    def body(h_r, s_r, c_r, w1_r, b1_r, w2_r, b2_r, v_r):
        agg = s_r[...] / jnp.maximum(c_r[...], 1.0)
        u = jnp.concatenate([h_r[...], agg], axis=-1)
        u = _swish(jnp.dot(u, w1_r[...], preferred_element_type=F32)
                   + b1_r[...])
        v_r[...] = jnp.dot(u, w2_r[...], preferred_element_type=F32) + b2_r[...]

    return pl.pallas_call(
        body, out_shape=jax.ShapeDtypeStruct((N, H), F32),
    )(h, segsum, cnt, Wu1, bu1r, Wu2, bu2r)


def _tc_bn(v, mean, var, gam, bet):
    def body(v_r, m_r, s_r, g_r, b_r, o_r):
        o_r[...] = ((v_r[...] - m_r[...]) / jnp.sqrt(s_r[...] + 1e-5)
                    * g_r[...] + b_r[...])

    return pl.pallas_call(
        body, out_shape=jax.ShapeDtypeStruct((N, H), F32),
    )(v, mean, var, gam, bet)


def _tc_bn_head(v, mean, var, gam, bet, Whead, bhead11):
    def body(v_r, m_r, s_r, g_r, b_r, w_r, bh_r, o_r):
        hn = ((v_r[...] - m_r[...]) / jnp.sqrt(s_r[...] + 1e-5)
              * g_r[...] + b_r[...])
        o_r[...] = (jnp.dot(hn, w_r[...], preferred_element_type=F32)
                    + bh_r[...])

    return pl.pallas_call(
        body, out_shape=jax.ShapeDtypeStruct((N, 1), F32),
    )(v, mean, var, gam, bet, Whead, bhead11)


def kernel(x, pos, edge_index, cell_offsets, Wemb, bemb, Wm1, bm1, Wm2, bm2,
           Wu1, bu1, Wu2, bu2, gamma, beta, Whead, bhead):
    src = edge_index[0].astype(jnp.int32)
    dst = edge_index[1].astype(jnp.int32)
    nl = Wm1.shape[0]

    e8 = _sc_geom(pos[:, 0], pos[:, 1], pos[:, 2], src, dst,
                  cell_offsets[:, 0], cell_offsets[:, 1],
                  cell_offsets[:, 2]).reshape(E, 8)
    rel = e8[:, :3]
    dist = jnp.linalg.norm(rel, axis=-1, keepdims=True)
    sh = jnp.concatenate([jnp.ones(rel.shape[:-1] + (1,), rel.dtype),
                          jnp.sqrt(3.0) * rel], axis=-1)
    geom5 = jnp.concatenate([dist, sh], axis=-1)
    cnt = jax.ops.segment_sum(jnp.ones((E, 1), F32), dst, num_segments=N)

    h = _tc_embed(x, Wemb, bemb.reshape(1, H))
    for l in range(nl):
        h_i, h_j = _sc_gather(h, src, dst)
        m2 = _tc_edge(h_i, h_j, geom5, Wm1[l], bm1[l].reshape(1, H),
                      Wm2[l], bm2[l].reshape(1, H))
        segsum = jax.ops.segment_sum(m2, dst, num_segments=N)
        v = _tc_update(h, segsum, cnt, Wu1[l], bu1[l].reshape(1, H),
                       Wu2[l], bu2[l].reshape(1, H))
        mean = jnp.mean(v, axis=0, keepdims=True)
        var = jnp.var(v, axis=0, keepdims=True)
        if l + 1 < nl:
            h = _tc_bn(v, mean, var, gamma[l].reshape(1, H),
                       beta[l].reshape(1, H))
        else:
            ne = _tc_bn_head(v, mean, var, gamma[l].reshape(1, H),
                             beta[l].reshape(1, H), Whead, bhead.reshape(1, 1))
    return jnp.sum(ne, axis=0)
